# baseline (speedup 1.0000x reference)
import functools

import jax
import jax.numpy as jnp
from jax import lax
from jax.experimental import pallas as pl
from jax.experimental.pallas import tpu as pltpu
from jax.experimental.pallas import tpu_sc as plsc

B = 16384
C = 1000
NW = 32
BPW = B // NW
CR = 32
CHW = CR * C
NCH = BPW // CR
G = CR // 16
UNROLL = 8


def _sc_body(x_hbm, y_hbm, loss_hbm, y_v, loss_v, buf0, buf1, sem0, sem1):
    wid = lax.axis_index("s") * 2 + lax.axis_index("c")
    base = wid * BPW

    pltpu.sync_copy(y_hbm.at[pl.ds(base, BPW)], y_v)

    pltpu.async_copy(x_hbm.at[pl.ds(base * C, CHW)], buf0, sem0)
    pltpu.async_copy(x_hbm.at[pl.ds(base * C + CHW, CHW)], buf1, sem1)

    lanes = lax.broadcasted_iota(jnp.int32, (16,), 0)

    def do_chunk(c, buf, sem):
        pltpu.make_async_copy(x_hbm.at[pl.ds(base * C, CHW)], buf, sem).wait()
        for g in range(G):
            lr0 = c * CR + g * 16
            rowoff = (lanes + g * 16) * C
            y16 = y_v[pl.ds(lr0, 16)]
            s16 = plsc.load_gather(buf, [rowoff + y16])
            s1 = s16 - 1.0

            def col_step(u, acc):
                for k in range(UNROLL):
                    col = u * UNROLL + k
                    v = plsc.load_gather(buf, [rowoff + col])
                    acc = acc + jnp.maximum(v - s1, 0.0)
                return acc

            acc = lax.fori_loop(0, C // UNROLL, col_step,
                                jnp.zeros((16,), jnp.float32))
            loss_v[pl.ds(lr0, 16)] = (acc - 1.0) * (1.0 / C)
        nxt = c + 2

        @pl.when(nxt < NCH)
        def _():
            pltpu.async_copy(
                x_hbm.at[pl.ds((base + nxt * CR) * C, CHW)], buf, sem)

    def pair(p, _):
        do_chunk(2 * p, buf0, sem0)
        do_chunk(2 * p + 1, buf1, sem1)
        return 0

    lax.fori_loop(0, NCH // 2, pair, 0)
    pltpu.sync_copy(loss_v, loss_hbm.at[pl.ds(base, BPW)])


@functools.partial(
    pl.kernel,
    mesh=plsc.VectorSubcoreMesh(core_axis_name="c", subcore_axis_name="s"),
    out_type=jax.ShapeDtypeStruct((B,), jnp.float32),
    compiler_params=pltpu.CompilerParams(
        use_tc_tiling_on_sc=False, needs_layout_passes=False),
    scratch_types=[
        pltpu.VMEM((BPW,), jnp.int32),
        pltpu.VMEM((BPW,), jnp.float32),
        pltpu.VMEM((CHW,), jnp.float32),
        pltpu.VMEM((CHW,), jnp.float32),
        pltpu.SemaphoreType.DMA,
        pltpu.SemaphoreType.DMA,
    ],
)
def _sc_hinge(x_hbm, y_hbm, loss_hbm, y_v, loss_v, buf0, buf1, sem0, sem1):
    _sc_body(x_hbm, y_hbm, loss_hbm, y_v, loss_v, buf0, buf1, sem0, sem1)


def kernel(output, y):
    return _sc_hinge(output.reshape(-1), y)


# speedup vs baseline: 1.0874x; 1.0874x over previous
import functools

import jax
import jax.numpy as jnp
from jax import lax
from jax.experimental import pallas as pl
from jax.experimental.pallas import tpu as pltpu
from jax.experimental.pallas import tpu_sc as plsc

B = 16384
C = 1000
NW = 32
BPW = B // NW
CR = 32
CHW = CR * C
NCH = BPW // CR
G = CR // 16
UNROLL = 8


def _sc_body(x_hbm, y_hbm, loss_hbm, y_v, loss_v, buf0, buf1, sem0, sem1):
    wid = lax.axis_index("s") * 2 + lax.axis_index("c")
    base = wid * BPW

    pltpu.sync_copy(y_hbm.at[pl.ds(base, BPW)], y_v)

    pltpu.async_copy(x_hbm.at[pl.ds(base * C, CHW)], buf0, sem0)
    pltpu.async_copy(x_hbm.at[pl.ds(base * C + CHW, CHW)], buf1, sem1)

    lanes = lax.broadcasted_iota(jnp.int32, (16,), 0)

    def do_chunk(c, buf, sem):
        pltpu.make_async_copy(x_hbm.at[pl.ds(base * C, CHW)], buf, sem).wait()
        for g in range(G):
            lr0 = c * CR + g * 16
            rowoff = (lanes + g * 16) * C
            y16 = y_v[pl.ds(lr0, 16)]
            s16 = plsc.load_gather(buf, [rowoff + y16])
            s1 = s16 - 1.0

            def col_step(u, accs):
                out = list(accs)
                for k in range(UNROLL):
                    col = u * UNROLL + k
                    v = plsc.load_gather(buf, [rowoff + col])
                    out[k % 4] = out[k % 4] + jnp.maximum(v - s1, 0.0)
                return tuple(out)

            zeros = jnp.zeros((16,), jnp.float32)
            a0, a1, a2, a3 = lax.fori_loop(
                0, C // UNROLL, col_step, (zeros, zeros, zeros, zeros))
            acc = (a0 + a1) + (a2 + a3)
            loss_v[pl.ds(lr0, 16)] = (acc - 1.0) * (1.0 / C)
        nxt = c + 2

        @pl.when(nxt < NCH)
        def _():
            pltpu.async_copy(
                x_hbm.at[pl.ds((base + nxt * CR) * C, CHW)], buf, sem)

    def pair(p, _):
        do_chunk(2 * p, buf0, sem0)
        do_chunk(2 * p + 1, buf1, sem1)
        return 0

    lax.fori_loop(0, NCH // 2, pair, 0)
    pltpu.sync_copy(loss_v, loss_hbm.at[pl.ds(base, BPW)])


@functools.partial(
    pl.kernel,
    mesh=plsc.VectorSubcoreMesh(core_axis_name="c", subcore_axis_name="s"),
    out_type=jax.ShapeDtypeStruct((B,), jnp.float32),
    compiler_params=pltpu.CompilerParams(
        use_tc_tiling_on_sc=False, needs_layout_passes=False),
    scratch_types=[
        pltpu.VMEM((BPW,), jnp.int32),
        pltpu.VMEM((BPW,), jnp.float32),
        pltpu.VMEM((CHW,), jnp.float32),
        pltpu.VMEM((CHW,), jnp.float32),
        pltpu.SemaphoreType.DMA,
        pltpu.SemaphoreType.DMA,
    ],
)
def _sc_hinge(x_hbm, y_hbm, loss_hbm, y_v, loss_v, buf0, buf1, sem0, sem1):
    _sc_body(x_hbm, y_hbm, loss_hbm, y_v, loss_v, buf0, buf1, sem0, sem1)


def kernel(output, y):
    return _sc_hinge(output.reshape(-1), y)


# speedup vs baseline: 1.0964x; 1.0083x over previous
import functools

import jax
import jax.numpy as jnp
from jax import lax
from jax.experimental import pallas as pl
from jax.experimental.pallas import tpu as pltpu
from jax.experimental.pallas import tpu_sc as plsc

B = 16384
C = 1000
NW = 32
BPW = B // NW
CR = 32
CHW = CR * C
NCH = BPW // CR
G = CR // 16
UNROLL = 8


def _sc_body(x_hbm, y_hbm, loss_hbm, y_v, loss_v, buf0, buf1, sem0, sem1):
    wid = lax.axis_index("s") * 2 + lax.axis_index("c")
    base = wid * BPW

    pltpu.sync_copy(y_hbm.at[pl.ds(base, BPW)], y_v)

    pltpu.async_copy(x_hbm.at[pl.ds(base * C, CHW)], buf0, sem0)
    pltpu.async_copy(x_hbm.at[pl.ds(base * C + CHW, CHW)], buf1, sem1)

    lanes = lax.broadcasted_iota(jnp.int32, (16,), 0)

    def do_chunk(c, buf, sem):
        pltpu.make_async_copy(x_hbm.at[pl.ds(base * C, CHW)], buf, sem).wait()
        zeros = jnp.zeros((16,), jnp.float32)
        for g in range(G):
            lr0 = c * CR + g * 16
            rowoff = (lanes + g * 16) * C
            y16 = y_v[pl.ds(lr0, 16)]
            s16 = plsc.load_gather(buf, [rowoff + y16])

            def row_body(r, sums16):
                rb = (g * 16 + r) * C
                s1 = jnp.sum(jnp.where(lanes == r, s16, 0.0)) - 1.0
                accs = [zeros, zeros, zeros, zeros]
                for i in range(C // 16):
                    v = buf[pl.ds(rb + i * 16, 16)]
                    accs[i % 4] = accs[i % 4] + jnp.maximum(v - s1, 0.0)
                v = buf[pl.ds(rb + (C - 16), 16)]
                t = jnp.maximum(v - s1, 0.0)
                accs[3] = accs[3] + jnp.where(lanes >= 16 - C % 16, t, 0.0)
                acc = (accs[0] + accs[1]) + (accs[2] + accs[3])
                total = jnp.sum(acc)
                return sums16 + jnp.where(lanes == r, total, 0.0)

            sums16 = lax.fori_loop(0, 16, row_body, zeros)
            loss_v[pl.ds(lr0, 16)] = (sums16 - 1.0) * (1.0 / C)
        nxt = c + 2

        @pl.when(nxt < NCH)
        def _():
            pltpu.async_copy(
                x_hbm.at[pl.ds((base + nxt * CR) * C, CHW)], buf, sem)

    def pair(p, _):
        do_chunk(2 * p, buf0, sem0)
        do_chunk(2 * p + 1, buf1, sem1)
        return 0

    lax.fori_loop(0, NCH // 2, pair, 0)
    pltpu.sync_copy(loss_v, loss_hbm.at[pl.ds(base, BPW)])


@functools.partial(
    pl.kernel,
    mesh=plsc.VectorSubcoreMesh(core_axis_name="c", subcore_axis_name="s"),
    out_type=jax.ShapeDtypeStruct((B,), jnp.float32),
    compiler_params=pltpu.CompilerParams(
        use_tc_tiling_on_sc=False, needs_layout_passes=False),
    scratch_types=[
        pltpu.VMEM((BPW,), jnp.int32),
        pltpu.VMEM((BPW,), jnp.float32),
        pltpu.VMEM((CHW,), jnp.float32),
        pltpu.VMEM((CHW,), jnp.float32),
        pltpu.SemaphoreType.DMA,
        pltpu.SemaphoreType.DMA,
    ],
)
def _sc_hinge(x_hbm, y_hbm, loss_hbm, y_v, loss_v, buf0, buf1, sem0, sem1):
    _sc_body(x_hbm, y_hbm, loss_hbm, y_v, loss_v, buf0, buf1, sem0, sem1)


def kernel(output, y):
    return _sc_hinge(output.reshape(-1), y)
